# baseline (speedup 1.0000x reference)
import jax
import jax.numpy as jnp
from jax.experimental import pallas as pl
from jax.experimental.pallas import tpu as pltpu

_K = 1024
_D = 64
_CH = 128
_ROWS = 2304


def _vq_kernel(x_ref, e_ref, o_ref, a_ref):
    e = e_ref[...]
    e2 = jnp.sum(e * e, axis=1, keepdims=True)
    a_ref[...] = jnp.concatenate(
        [-2.0 * e, e2, jnp.zeros((_K, 128 - _D - 1), jnp.float32)], axis=1)

    x = x_ref[...]
    xa = jnp.concatenate(
        [x, jnp.ones((_ROWS, 128 - _D), jnp.float32)], axis=1)
    rmin = ridx = None
    for c in range(_K // _CH):
        a_c = a_ref[c * _CH:(c + 1) * _CH, :]
        s = jax.lax.dot_general(
            a_c, xa, (((1,), (1,)), ((), ())),
            preferred_element_type=jnp.float32,
            precision=jax.lax.Precision.HIGHEST)
        m = jnp.min(s, axis=0, keepdims=True)
        ids = jax.lax.broadcasted_iota(jnp.int32, s.shape, 0) + c * _CH
        idx = jnp.min(jnp.where(s == m, ids, _K), axis=0, keepdims=True)
        if rmin is None:
            rmin, ridx = m, idx
        else:
            upd = m < rmin
            rmin = jnp.where(upd, m, rmin)
            ridx = jnp.where(upd, idx, ridx)
    o_ref[0, 0, :] = ridx[0]


def kernel(input, embedding):
    B, w, h, c = input.shape
    n = B * w * h
    flat = input.reshape(n, c)
    nblk = n // _ROWS
    out = pl.pallas_call(
        _vq_kernel,
        grid=(nblk,),
        in_specs=[
            pl.BlockSpec((_ROWS, _D), lambda i: (i, 0)),
            pl.BlockSpec((_K, _D), lambda i: (0, 0)),
        ],
        out_specs=pl.BlockSpec((1, 1, _ROWS), lambda i: (i, 0, 0)),
        out_shape=jax.ShapeDtypeStruct((nblk, 1, _ROWS), jnp.int32),
        scratch_shapes=[pltpu.VMEM((_K, 128), jnp.float32)],
        compiler_params=pltpu.CompilerParams(
            dimension_semantics=("parallel",)),
    )(flat, embedding)
    return out.reshape(B, w, h)


# speedup vs baseline: 1.0035x; 1.0035x over previous
import jax
import jax.numpy as jnp
from jax.experimental import pallas as pl
from jax.experimental.pallas import tpu as pltpu

_K = 1024
_D = 64
_CH = 128
_ROWS = 4608


def _vq_kernel(x_ref, e_ref, o_ref, a_ref):
    @pl.when(pl.program_id(0) == 0)
    def _():
        e = e_ref[...]
        e2 = jnp.sum(e * e, axis=1, keepdims=True)
        a_ref[...] = jnp.concatenate(
            [-2.0 * e, e2, jnp.zeros((_K, 128 - _D - 1), jnp.float32)], axis=1)

    x = x_ref[...]
    xa = jnp.concatenate(
        [x, jnp.ones((_ROWS, 128 - _D), jnp.float32)], axis=1)
    rmin = ridx = None
    for c in range(_K // _CH):
        a_c = a_ref[c * _CH:(c + 1) * _CH, :]
        s = jax.lax.dot_general(
            a_c, xa, (((1,), (1,)), ((), ())),
            preferred_element_type=jnp.float32,
            precision=jax.lax.Precision.HIGHEST)
        m = jnp.min(s, axis=0, keepdims=True)
        ids = jax.lax.broadcasted_iota(jnp.int32, s.shape, 0) + c * _CH
        idx = jnp.min(jnp.where(s == m, ids, _K), axis=0, keepdims=True)
        if rmin is None:
            rmin, ridx = m, idx
        else:
            upd = m < rmin
            rmin = jnp.where(upd, m, rmin)
            ridx = jnp.where(upd, idx, ridx)
    o_ref[0, 0, :] = ridx[0]


def kernel(input, embedding):
    B, w, h, c = input.shape
    n = B * w * h
    flat = input.reshape(n, c)
    nblk = n // _ROWS
    out = pl.pallas_call(
        _vq_kernel,
        grid=(nblk,),
        in_specs=[
            pl.BlockSpec((_ROWS, _D), lambda i: (i, 0)),
            pl.BlockSpec((_K, _D), lambda i: (0, 0)),
        ],
        out_specs=pl.BlockSpec((1, 1, _ROWS), lambda i: (i, 0, 0)),
        out_shape=jax.ShapeDtypeStruct((nblk, 1, _ROWS), jnp.int32),
        scratch_shapes=[pltpu.VMEM((_K, 128), jnp.float32)],
    )(flat, embedding)
    return out.reshape(B, w, h)
